# baseline (speedup 1.0000x reference)
import functools

import jax
import jax.numpy as jnp
from jax import lax
from jax.experimental import pallas as pl
from jax.experimental.pallas import tpu as pltpu
from jax.experimental.pallas import tpu_sc as plsc

B, T, D = 4, 2048, 768
V, K = 100000, 100
EPS = 1e-12

NC, NS = 2, 16
NW = NC * NS
N_TOK = B * T

STAGES = ((0, 1024), (1024, 512), (1536, 512))
CHUNK = 64


@functools.lru_cache(maxsize=None)
def _get_sc_gather(n_tok_s):
    rows_per_w = n_tok_s // NW
    nchunk = rows_per_w // CHUNK
    mesh = plsc.VectorSubcoreMesh(
        core_axis_name="c", subcore_axis_name="s",
        num_cores=NC, num_subcores=NS,
    )

    @functools.partial(
        pl.kernel,
        out_type=jax.ShapeDtypeStruct((n_tok_s, D), jnp.float32),
        mesh=mesh,
        scratch_types=[
            pltpu.VMEM((nchunk, CHUNK), jnp.int32),
            pltpu.VMEM((2, CHUNK, D), jnp.float32),
            pltpu.SemaphoreType.DMA,
            pltpu.SemaphoreType.DMA,
        ],
    )
    def _sc_gather(tok_hbm, idx_hbm, out_hbm, idx_v, rows_v, sem0, sem1):
        wid = lax.axis_index("s") * NC + lax.axis_index("c")
        base = wid * rows_per_w
        sems = (sem0, sem1)
        pltpu.sync_copy(idx_hbm.at[wid], idx_v)
        copies = [None] * nchunk
        copies[0] = pltpu.async_copy(
            tok_hbm.at[idx_v.at[0]], rows_v.at[0], sems[0])
        for c in range(nchunk):
            if c + 1 < nchunk:
                nb = (c + 1) % 2
                copies[c + 1] = pltpu.async_copy(
                    tok_hbm.at[idx_v.at[c + 1]], rows_v.at[nb], sems[nb]
                )
            copies[c].wait()
            pltpu.sync_copy(
                rows_v.at[c % 2], out_hbm.at[pl.ds(base + c * CHUNK, CHUNK)]
            )

    return _sc_gather


BT = 1024


def _tc_body_first(gath_ref, p_ref, s_ref, lang_ref, sw_ref, pos_ref,
                   g_ref, b_ref, out_ref):
    x = gath_ref[...] + pos_ref[...]
    x = x + jnp.dot(p_ref[...], lang_ref[...],
                    preferred_element_type=jnp.float32)
    x = x + s_ref[...] * sw_ref[...]
    mu = jnp.mean(x, axis=1, keepdims=True)
    xc = x - mu
    var = jnp.mean(xc * xc, axis=1, keepdims=True)
    out_ref[...] = xc * lax.rsqrt(var + EPS) * g_ref[...] + b_ref[...]


def _tc_body_rest(gath_ref, p_ref, s_ref, lang_ref, sw_ref, pos_ref,
                  g_ref, b_ref, prev_ref, out_ref):
    del prev_ref
    _tc_body_first(gath_ref, p_ref, s_ref, lang_ref, sw_ref, pos_ref,
                   g_ref, b_ref, out_ref)


def _make_tc(stage_idx, t0, tl):
    bt = min(BT, tl)
    nb = tl // bt
    tb0 = t0 // bt

    def full(g):
        return ((g // nb) * (T // bt) + tb0 + g % nb, 0)

    def half(g):
        return (g, 0)

    def fixed(g):
        return (0, 0)

    in_specs = [
        pl.BlockSpec((bt, D), half),
        pl.BlockSpec((bt, K), full),
        pl.BlockSpec((bt, 1), full),
        pl.BlockSpec((K, D), fixed),
        pl.BlockSpec((1, D), fixed),
        pl.BlockSpec((bt, D), lambda g: (tb0 + g % nb, 0)),
        pl.BlockSpec((1, D), fixed),
        pl.BlockSpec((1, D), fixed),
    ]
    kwargs = {}
    body = _tc_body_first
    if stage_idx > 0:
        in_specs.append(pl.BlockSpec(memory_space=pl.ANY))
        kwargs["input_output_aliases"] = {8: 0}
        body = _tc_body_rest
    return pl.pallas_call(
        body,
        grid=(B * nb,),
        in_specs=in_specs,
        out_specs=pl.BlockSpec((bt, D), full),
        out_shape=jax.ShapeDtypeStruct((N_TOK, D), jnp.float32),
        **kwargs,
    )


def kernel(input_ids, p, s, tok_table, pos_table, lang_table, switch_emb,
           ln_gamma, ln_beta):
    ids = input_ids.astype(jnp.int32)
    p2 = p.reshape(N_TOK, K)
    s2 = s.reshape(N_TOK, 1)
    sw = switch_emb[None, :]
    g2 = ln_gamma[None, :]
    b2 = ln_beta[None, :]
    out = None
    for si, (t0, tl) in enumerate(STAGES):
        n_tok_s = B * tl
        nchunk = (n_tok_s // NW) // CHUNK
        ids_s = ids[:, t0:t0 + tl].reshape(NW, nchunk, CHUNK)
        gath = _get_sc_gather(n_tok_s)(tok_table, ids_s)
        args = [gath, p2, s2, lang_table, sw, pos_table, g2, b2]
        if si > 0:
            args.append(out)
        out = _make_tc(si, t0, tl)(*args)
    return out.reshape(B, T, D)


# speedup vs baseline: 1.0512x; 1.0512x over previous
import functools

import jax
import jax.numpy as jnp
from jax import lax
from jax.experimental import pallas as pl
from jax.experimental.pallas import tpu as pltpu
from jax.experimental.pallas import tpu_sc as plsc

B, T, D = 4, 2048, 768
V, K = 100000, 100
EPS = 1e-12

NC, NS = 2, 16
NW = NC * NS
N_TOK = B * T

STAGES = ((0, 1024), (1024, 1024))
CHUNK = 64


@functools.lru_cache(maxsize=None)
def _get_sc_gather(n_tok_s):
    rows_per_w = n_tok_s // NW
    nchunk = rows_per_w // CHUNK
    mesh = plsc.VectorSubcoreMesh(
        core_axis_name="c", subcore_axis_name="s",
        num_cores=NC, num_subcores=NS,
    )

    @functools.partial(
        pl.kernel,
        out_type=jax.ShapeDtypeStruct((n_tok_s, D), jnp.float32),
        mesh=mesh,
        scratch_types=[
            pltpu.VMEM((nchunk, CHUNK), jnp.int32),
            pltpu.VMEM((2, CHUNK, D), jnp.float32),
            pltpu.SemaphoreType.DMA,
            pltpu.SemaphoreType.DMA,
        ],
    )
    def _sc_gather(tok_hbm, idx_hbm, out_hbm, idx_v, rows_v, sem0, sem1):
        wid = lax.axis_index("s") * NC + lax.axis_index("c")
        base = wid * rows_per_w
        sems = (sem0, sem1)
        pltpu.sync_copy(idx_hbm.at[wid], idx_v)
        copies = [None] * nchunk
        copies[0] = pltpu.async_copy(
            tok_hbm.at[idx_v.at[0]], rows_v.at[0], sems[0])
        for c in range(nchunk):
            if c + 1 < nchunk:
                nb = (c + 1) % 2
                copies[c + 1] = pltpu.async_copy(
                    tok_hbm.at[idx_v.at[c + 1]], rows_v.at[nb], sems[nb]
                )
            copies[c].wait()
            pltpu.sync_copy(
                rows_v.at[c % 2], out_hbm.at[pl.ds(base + c * CHUNK, CHUNK)]
            )

    return _sc_gather


BT = 1024


def _tc_body_first(gath_ref, p_ref, s_ref, lang_ref, sw_ref, pos_ref,
                   g_ref, b_ref, out_ref):
    x = gath_ref[...] + pos_ref[...]
    x = x + jnp.dot(p_ref[...], lang_ref[...],
                    preferred_element_type=jnp.float32)
    x = x + s_ref[...] * sw_ref[...]
    mu = jnp.mean(x, axis=1, keepdims=True)
    xc = x - mu
    var = jnp.mean(xc * xc, axis=1, keepdims=True)
    out_ref[...] = xc * lax.rsqrt(var + EPS) * g_ref[...] + b_ref[...]


def _tc_body_rest(gath_ref, p_ref, s_ref, lang_ref, sw_ref, pos_ref,
                  g_ref, b_ref, prev_ref, out_ref):
    del prev_ref
    _tc_body_first(gath_ref, p_ref, s_ref, lang_ref, sw_ref, pos_ref,
                   g_ref, b_ref, out_ref)


def _make_tc(stage_idx, t0, tl):
    bt = min(BT, tl)
    nb = tl // bt
    tb0 = t0 // bt

    def full(g):
        return ((g // nb) * (T // bt) + tb0 + g % nb, 0)

    def half(g):
        return (g, 0)

    def fixed(g):
        return (0, 0)

    in_specs = [
        pl.BlockSpec((bt, D), half),
        pl.BlockSpec((bt, K), full),
        pl.BlockSpec((bt, 1), full),
        pl.BlockSpec((K, D), fixed),
        pl.BlockSpec((1, D), fixed),
        pl.BlockSpec((bt, D), lambda g: (tb0 + g % nb, 0)),
        pl.BlockSpec((1, D), fixed),
        pl.BlockSpec((1, D), fixed),
    ]
    kwargs = {}
    body = _tc_body_first
    if stage_idx > 0:
        in_specs.append(pl.BlockSpec(memory_space=pl.ANY))
        kwargs["input_output_aliases"] = {8: 0}
        body = _tc_body_rest
    return pl.pallas_call(
        body,
        grid=(B * nb,),
        in_specs=in_specs,
        out_specs=pl.BlockSpec((bt, D), full),
        out_shape=jax.ShapeDtypeStruct((N_TOK, D), jnp.float32),
        **kwargs,
    )


def kernel(input_ids, p, s, tok_table, pos_table, lang_table, switch_emb,
           ln_gamma, ln_beta):
    ids = input_ids.astype(jnp.int32)
    p2 = p.reshape(N_TOK, K)
    s2 = s.reshape(N_TOK, 1)
    sw = switch_emb[None, :]
    g2 = ln_gamma[None, :]
    b2 = ln_beta[None, :]
    out = None
    for si, (t0, tl) in enumerate(STAGES):
        n_tok_s = B * tl
        nchunk = (n_tok_s // NW) // CHUNK
        ids_s = ids[:, t0:t0 + tl].reshape(NW, nchunk, CHUNK)
        gath = _get_sc_gather(n_tok_s)(tok_table, ids_s)
        args = [gath, p2, s2, lang_table, sw, pos_table, g2, b2]
        if si > 0:
            args.append(out)
        out = _make_tc(si, t0, tl)(*args)
    return out.reshape(B, T, D)
